# baseline (speedup 1.0000x reference)
import functools

import jax
import jax.numpy as jnp
from jax import lax
from jax.experimental import pallas as pl
from jax.experimental.pallas import tpu as pltpu
from jax.experimental.pallas import tpu_sc as plsc

_CHUNK = 128
_NBUF = 4
_L = 16


def _lookup(idx2, table, v, d):
    n_rows, chunk = idx2.shape
    info = plsc.get_sparse_core_info()
    nw = info.num_cores * info.num_subcores
    n_chunks = n_rows // nw
    b_per_w = n_chunks * chunk
    nbuf = min(_NBUF, n_chunks)
    n_col = d // _L
    n_grp = chunk // _L

    mesh = plsc.VectorSubcoreMesh(core_axis_name="c", subcore_axis_name="s")

    @functools.partial(
        pl.kernel,
        mesh=mesh,
        compiler_params=pltpu.CompilerParams(needs_layout_passes=False),
        out_type=jax.ShapeDtypeStruct((n_rows * chunk, d), jnp.float32),
        scratch_types=(
            [pltpu.VMEM((v * d,), jnp.float32)]
            + [pltpu.VMEM((n_chunks, chunk), jnp.int32)]
            + [pltpu.VMEM((nbuf * chunk, d), jnp.float32)]
            + [pltpu.SemaphoreType.DMA((nbuf,))]
        ),
    )
    def k(table_hbm, idx_hbm, out_hbm, table_v, idx_v, buf, sems):
        wid = lax.axis_index("s") * info.num_cores + lax.axis_index("c")
        pltpu.sync_copy(table_hbm, table_v)
        pltpu.sync_copy(idx_hbm.at[pl.ds(wid * n_chunks, n_chunks)], idx_v)

        cols = [jnp.arange(_L, dtype=jnp.int32) + _L * j for j in range(n_col)]
        lane = [jnp.full((_L,), r, jnp.int32) for r in range(_L)]
        out_base = wid * b_per_w

        def chunk_body(c, _):
            b = lax.rem(c, nbuf)
            slot = b * chunk

            @pl.when(c >= nbuf)
            def _wait_slot():
                pltpu.make_async_copy(
                    buf.at[pl.ds(slot, chunk)],
                    out_hbm.at[pl.ds(out_base, chunk)],
                    sems.at[b],
                ).wait()

            def group_body(g):
                idxvec = idx_v[c, pl.ds(g * _L, _L)]
                row0 = slot + g * _L
                for r in range(_L):
                    iv = idxvec.at[lane[r]].get(mode="promise_in_bounds")
                    ivd = iv * d
                    for j in range(n_col):
                        vals = plsc.load_gather(table_v, [ivd + cols[j]])
                        buf[row0 + r, pl.ds(j * _L, _L)] = vals

            plsc.parallel_loop(0, n_grp, unroll=8)(group_body)
            pltpu.async_copy(
                buf.at[pl.ds(slot, chunk)],
                out_hbm.at[pl.ds(out_base + c * chunk, chunk)],
                sems.at[b],
            )
            return 0

        lax.fori_loop(0, n_chunks, chunk_body, 0)
        for b in range(nbuf):
            pltpu.make_async_copy(
                buf.at[pl.ds(b * chunk, chunk)],
                out_hbm.at[pl.ds(out_base, chunk)],
                sems.at[b],
            ).wait()

    return k(table, idx2)


def kernel(input, table):
    v, d = table.shape
    idx = input.reshape(-1).astype(jnp.int32)
    idx2 = idx.reshape(-1, _CHUNK)
    out = _lookup(idx2, table.astype(jnp.float32).reshape(-1), v, d)
    return out.reshape(input.shape + (d,))


# speedup vs baseline: 3.7684x; 3.7684x over previous
import functools

import jax
import jax.numpy as jnp
from jax import lax
from jax.experimental import pallas as pl
from jax.experimental.pallas import tpu as pltpu
from jax.experimental.pallas import tpu_sc as plsc

_CHUNK = 128
_NBUF = 4


def _lookup(idx2, table):
    n_rows, chunk = idx2.shape
    v, d = table.shape
    info = plsc.get_sparse_core_info()
    nw = info.num_cores * info.num_subcores
    n_chunks = n_rows // nw
    b_per_w = n_chunks * chunk
    nbuf = min(_NBUF, n_chunks)

    mesh = plsc.VectorSubcoreMesh(core_axis_name="c", subcore_axis_name="s")

    @functools.partial(
        pl.kernel,
        mesh=mesh,
        out_type=jax.ShapeDtypeStruct((n_rows * chunk, d), jnp.float32),
        scratch_types=(
            [pltpu.VMEM_SHARED((v, d), jnp.float32)]
            + [pltpu.VMEM((n_chunks, chunk), jnp.int32)]
            + [pltpu.VMEM((chunk, d), jnp.float32) for _ in range(nbuf)]
            + [pltpu.SemaphoreType.DMA for _ in range(2 * nbuf)]
        ),
    )
    def k(table_hbm, idx_hbm, out_hbm, table_sh, idx_v, *rest):
        bufs = rest[:nbuf]
        sems_g = rest[nbuf : 2 * nbuf]
        sems_s = rest[2 * nbuf : 3 * nbuf]
        sid = lax.axis_index("s")
        wid = sid * info.num_cores + lax.axis_index("c")
        @pl.when(sid == 0)
        def _():
            pltpu.sync_copy(table_hbm, table_sh)

        pltpu.sync_copy(idx_hbm.at[pl.ds(wid * n_chunks, n_chunks)], idx_v)
        plsc.subcore_barrier()

        gath = {}
        scat = {}

        def start_gather(c):
            b = c % nbuf
            gath[c] = pltpu.async_copy(
                table_sh.at[idx_v.at[c]], bufs[b], sems_g[b]
            )

        for c in range(nbuf):
            start_gather(c)
        out_base = wid * b_per_w
        for c in range(n_chunks):
            b = c % nbuf
            gath[c].wait()
            scat[c] = pltpu.async_copy(
                bufs[b], out_hbm.at[pl.ds(out_base + c * chunk, chunk)], sems_s[b]
            )
            nxt = c + nbuf
            if nxt < n_chunks:
                scat[c].wait()
                start_gather(nxt)
        for c in range(n_chunks - nbuf, n_chunks):
            scat[c].wait()

    return k(table, idx2)


_TC_BLK = 2048


def _tc_lookup(idx, table):
    n, = idx.shape
    v, d = table.shape
    nb = n // _TC_BLK
    idx3 = idx.reshape(nb, 1, _TC_BLK)

    def body(idx_ref, table_ref, out_ref):
        idxb = idx_ref[0, 0, :]
        iota = lax.broadcasted_iota(jnp.int32, (_TC_BLK, v), 1)
        oh = (idxb[:, None] == iota).astype(jnp.float32)
        out_ref[...] = jnp.dot(
            oh, table_ref[...], preferred_element_type=jnp.float32
        )

    return pl.pallas_call(
        body,
        grid=(nb,),
        in_specs=[
            pl.BlockSpec((1, 1, _TC_BLK), lambda i: (i, 0, 0)),
            pl.BlockSpec((v, d), lambda i: (0, 0)),
        ],
        out_specs=pl.BlockSpec((_TC_BLK, d), lambda i: (i, 0)),
        out_shape=jax.ShapeDtypeStruct((n, d), jnp.float32),
    )(idx3, table)


def kernel(input, table):
    d = table.shape[-1]
    idx = input.reshape(-1).astype(jnp.int32)
    out = _tc_lookup(idx, table.astype(jnp.float32))
    return out.reshape(input.shape + (d,))
